# baseline (speedup 1.0000x reference)
import functools

import jax
import jax.numpy as jnp
from jax import lax
from jax.experimental import pallas as pl
from jax.experimental.pallas import tpu as pltpu
from jax.experimental.pallas import tpu_sc as plsc

_B = 16384
_D = 64
_H1 = 256
_H2 = 128
_PACK = 4
_V4 = 1000000 // _PACK

_NC = 2
_NS = 16
_NW = _NC * _NS
_ROWS_PER_W = _B // _NW
_CHUNK = 128
_NCHUNK = _ROWS_PER_W // _CHUNK


def _pack_table(table):
    t = lax.bitcast_convert_type(table, jnp.uint32)
    b = (t + jnp.uint32(0x7FFF) + ((t >> 16) & jnp.uint32(1))) >> 16
    w = b[:, 0::2] | (b[:, 1::2] << 16)
    return lax.bitcast_convert_type(w.reshape(_V4, 128), jnp.float32)


def _sc_gather(idx4, table3):
    mesh = plsc.VectorSubcoreMesh(core_axis_name="c", subcore_axis_name="s")

    @functools.partial(
        pl.kernel,
        out_type=jax.ShapeDtypeStruct((_B, 128), jnp.float32),
        mesh=mesh,
        scratch_types=[
            pltpu.VMEM((_NCHUNK, _CHUNK), jnp.int32),
            pltpu.VMEM((2, _CHUNK, 128), jnp.float32),
            pltpu.SemaphoreType.DMA((2,)),
            pltpu.SemaphoreType.DMA((2,)),
        ],
    )
    def k(tab_hbm, idx_hbm, out_hbm, idx_v, buf_v, gsem, wsem):
        wid = lax.axis_index("s") * _NC + lax.axis_index("c")
        pltpu.sync_copy(idx_hbm.at[pl.ds(wid * _NCHUNK, _NCHUNK)], idx_v)
        base = wid * _ROWS_PER_W

        def gather(j):
            s = j % 2
            return pltpu.async_copy(tab_hbm.at[idx_v.at[j]], buf_v.at[s],
                                    gsem.at[s])

        def writeback(j):
            s = j % 2
            dst = pl.ds(base + j * _CHUNK, _CHUNK)
            return pltpu.async_copy(buf_v.at[s], out_hbm.at[dst], wsem.at[s])

        g = {0: gather(0)}
        wb = {}
        for j in range(_NCHUNK):
            if j + 1 < _NCHUNK:
                if j - 1 >= 0:
                    wb[j - 1].wait()
                g[j + 1] = gather(j + 1)
            g[j].wait()
            wb[j] = writeback(j)
        wb[_NCHUNK - 2].wait()
        wb[_NCHUNK - 1].wait()

    return k(table3, idx4)


def _decode_quarter(rows_ref, par_ref):
    words = rows_ref[...]
    p = par_ref[...]
    lo = jnp.where(p < 1, words[:, 0:32], words[:, 32:64])
    hi = jnp.where(p < 3, words[:, 64:96], words[:, 96:128])
    w = lax.bitcast_convert_type(jnp.where(p < 2, lo, hi), jnp.int32)
    even = lax.bitcast_convert_type(w << 16, jnp.float32)
    odd = lax.bitcast_convert_type(w & jnp.int32(-65536), jnp.float32)
    return even, odd


def _mlp_body(u4_ref, i4_ref, up_ref, ip_ref, w1ue_ref, w1uo_ref, w1ie_ref,
              w1io_ref, b1_ref, w2_ref, b2_ref, wf_ref, bf_ref, o_ref):
    ue, uo = _decode_quarter(u4_ref, up_ref)
    ie, io = _decode_quarter(i4_ref, ip_ref)
    h1 = jnp.dot(ue, w1ue_ref[...], preferred_element_type=jnp.float32)
    h1 = h1 + jnp.dot(uo, w1uo_ref[...], preferred_element_type=jnp.float32)
    h1 = h1 + jnp.dot(ie, w1ie_ref[...], preferred_element_type=jnp.float32)
    h1 = h1 + jnp.dot(io, w1io_ref[...], preferred_element_type=jnp.float32)
    h1 = jnp.maximum(h1 + b1_ref[...], 0.0)
    h2 = jnp.dot(h1, w2_ref[...], preferred_element_type=jnp.float32)
    h2 = jnp.maximum(h2 + b2_ref[...], 0.0)
    z = jnp.dot(h2, wf_ref[...], preferred_element_type=jnp.float32)
    o_ref[...] = jax.nn.sigmoid(z + bf_ref[...])


def _mlp(u4, i4, uparity, iparity, W1, b1, W2, b2, Wf, bf, blk=2048):
    n_blocks = _B // blk
    wspec = pl.BlockSpec((32, _H1), lambda i: (0, 0))
    return pl.pallas_call(
        _mlp_body,
        grid=(n_blocks,),
        in_specs=[
            pl.BlockSpec((blk, 128), lambda i: (i, 0)),
            pl.BlockSpec((blk, 128), lambda i: (i, 0)),
            pl.BlockSpec((blk, 1), lambda i: (i, 0)),
            pl.BlockSpec((blk, 1), lambda i: (i, 0)),
            wspec, wspec, wspec, wspec,
            pl.BlockSpec((1, _H1), lambda i: (0, 0)),
            pl.BlockSpec((_H1, _H2), lambda i: (0, 0)),
            pl.BlockSpec((1, _H2), lambda i: (0, 0)),
            pl.BlockSpec((_H2, 1), lambda i: (0, 0)),
            pl.BlockSpec((1, 1), lambda i: (0, 0)),
        ],
        out_specs=pl.BlockSpec((blk, 1), lambda i: (i, 0)),
        out_shape=jax.ShapeDtypeStruct((_B, 1), jnp.float32),
    )(u4, i4, uparity, iparity, W1[0:_D:2], W1[1:_D:2], W1[_D::2],
      W1[_D + 1::2], b1.reshape(1, _H1), W2, b2.reshape(1, _H2), Wf,
      bf.reshape(1, 1))


def kernel(users, items, user_table, item_table, W1, b1, W2, b2, Wf, bf):
    users = users.astype(jnp.int32)
    items = items.astype(jnp.int32)
    uidx4 = (users // _PACK).reshape(_B // _CHUNK, _CHUNK)
    iidx4 = (items // _PACK).reshape(_B // _CHUNK, _CHUNK)
    uparity = (users % _PACK).reshape(_B, 1)
    iparity = (items % _PACK).reshape(_B, 1)
    ut3 = _pack_table(user_table)
    gu = _sc_gather(uidx4, ut3)
    it3 = _pack_table(item_table)
    gi = _sc_gather(iidx4, it3)
    return _mlp(gu, gi, uparity, iparity, W1, b1, W2, b2, Wf, bf)


# speedup vs baseline: 3.4280x; 3.4280x over previous
import functools

import jax
import jax.numpy as jnp
from jax import lax
from jax.experimental import pallas as pl
from jax.experimental.pallas import tpu as pltpu
from jax.experimental.pallas import tpu_sc as plsc

_B = 16384
_D = 64
_H1 = 256
_H2 = 128
_V2 = 1000000 // 2

_NC = 2
_NS = 16
_NW = _NC * _NS
_ROWS_PER_W = _B // _NW
_CHUNK = 128
_NCHUNK = _ROWS_PER_W // _CHUNK


def _sc_gather(idx2, table2):
    mesh = plsc.VectorSubcoreMesh(core_axis_name="c", subcore_axis_name="s")

    @functools.partial(
        pl.kernel,
        out_type=jax.ShapeDtypeStruct((_B, 128), jnp.float32),
        mesh=mesh,
        scratch_types=[
            pltpu.VMEM((_NCHUNK, _CHUNK), jnp.int32),
            pltpu.VMEM((2, _CHUNK, 128), jnp.float32),
            pltpu.SemaphoreType.DMA((2,)),
            pltpu.SemaphoreType.DMA((2,)),
        ],
    )
    def k(tab_hbm, idx_hbm, out_hbm, idx_v, buf_v, gsem, wsem):
        wid = lax.axis_index("s") * _NC + lax.axis_index("c")
        pltpu.sync_copy(idx_hbm.at[pl.ds(wid * _NCHUNK, _NCHUNK)], idx_v)
        base = wid * _ROWS_PER_W

        def gather(j):
            s = j % 2
            return pltpu.async_copy(tab_hbm.at[idx_v.at[j]], buf_v.at[s],
                                    gsem.at[s])

        def writeback(j):
            s = j % 2
            dst = pl.ds(base + j * _CHUNK, _CHUNK)
            return pltpu.async_copy(buf_v.at[s], out_hbm.at[dst], wsem.at[s])

        g = {0: gather(0)}
        wb = {}
        for j in range(_NCHUNK):
            if j + 1 < _NCHUNK:
                if j - 1 >= 0:
                    wb[j - 1].wait()
                g[j + 1] = gather(j + 1)
            g[j].wait()
            wb[j] = writeback(j)
        wb[_NCHUNK - 2].wait()
        wb[_NCHUNK - 1].wait()

    return k(table2, idx2)


def _mlp_body(u2_ref, i2_ref, up_ref, ip_ref, w1u_ref,
              w1i_ref, b1_ref, w2_ref, b2_ref, wf_ref, bf_ref, o_ref):
    u2 = u2_ref[...]
    i2 = i2_ref[...]
    u = jnp.where(up_ref[...] > 0, u2[:, _D:], u2[:, :_D])
    it = jnp.where(ip_ref[...] > 0, i2[:, _D:], i2[:, :_D])
    h1 = jnp.dot(u, w1u_ref[...], preferred_element_type=jnp.float32)
    h1 = h1 + jnp.dot(it, w1i_ref[...], preferred_element_type=jnp.float32)
    h1 = jnp.maximum(h1 + b1_ref[...], 0.0)
    h2 = jnp.dot(h1, w2_ref[...], preferred_element_type=jnp.float32)
    h2 = jnp.maximum(h2 + b2_ref[...], 0.0)
    z = jnp.dot(h2, wf_ref[...], preferred_element_type=jnp.float32)
    o_ref[...] = jax.nn.sigmoid(z + bf_ref[...])


def _mlp(u2, i2, uparity, iparity, W1, b1, W2, b2, Wf, bf, blk=2048):
    n_blocks = _B // blk
    return pl.pallas_call(
        _mlp_body,
        grid=(n_blocks,),
        in_specs=[
            pl.BlockSpec((blk, 128), lambda i: (i, 0)),
            pl.BlockSpec((blk, 128), lambda i: (i, 0)),
            pl.BlockSpec((blk, 1), lambda i: (i, 0)),
            pl.BlockSpec((blk, 1), lambda i: (i, 0)),
            pl.BlockSpec((_D, _H1), lambda i: (0, 0)),
            pl.BlockSpec((_D, _H1), lambda i: (0, 0)),
            pl.BlockSpec((1, _H1), lambda i: (0, 0)),
            pl.BlockSpec((_H1, _H2), lambda i: (0, 0)),
            pl.BlockSpec((1, _H2), lambda i: (0, 0)),
            pl.BlockSpec((_H2, 1), lambda i: (0, 0)),
            pl.BlockSpec((1, 1), lambda i: (0, 0)),
        ],
        out_specs=pl.BlockSpec((blk, 1), lambda i: (i, 0)),
        out_shape=jax.ShapeDtypeStruct((_B, 1), jnp.float32),
    )(u2, i2, uparity, iparity, W1[:_D], W1[_D:],
      b1.reshape(1, _H1), W2, b2.reshape(1, _H2), Wf, bf.reshape(1, 1))


def kernel(users, items, user_table, item_table, W1, b1, W2, b2, Wf, bf):
    users = users.astype(jnp.int32)
    items = items.astype(jnp.int32)
    uidx2 = (users // 2).reshape(_B // _CHUNK, _CHUNK)
    iidx2 = (items // 2).reshape(_B // _CHUNK, _CHUNK)
    uparity = (users % 2).reshape(_B, 1)
    iparity = (items % 2).reshape(_B, 1)
    ut2 = user_table.reshape(_V2, 2 * _D)
    gu = _sc_gather(uidx2, ut2)
    it2 = item_table.reshape(_V2, 2 * _D)
    gi = _sc_gather(iidx2, it2)
    return _mlp(gu, gi, uparity, iparity, W1, b1, W2, b2, Wf, bf)


# speedup vs baseline: 3.8317x; 1.1178x over previous
import functools

import jax
import jax.numpy as jnp
from jax import lax
from jax.experimental import pallas as pl
from jax.experimental.pallas import tpu as pltpu
from jax.experimental.pallas import tpu_sc as plsc

_B = 16384
_D = 64
_H1 = 256
_H2 = 128
_V = 1000000

_NC = 2
_NS = 16
_NW = _NC * _NS
_ROWS_PER_W = _B // _NW
_BATCH = 16


def _sc_gather(idx8, table):
    mesh = plsc.VectorSubcoreMesh(core_axis_name="c", subcore_axis_name="s")
    nbatch = _ROWS_PER_W // _BATCH

    @functools.partial(
        pl.kernel,
        out_type=jax.ShapeDtypeStruct((_B, 8, _D), jnp.bfloat16),
        mesh=mesh,
        scratch_types=[
            pltpu.VMEM((_ROWS_PER_W,), jnp.int32),
            pltpu.VMEM((2, _BATCH, 8, _D), jnp.bfloat16),
            pltpu.SemaphoreType.DMA,
            pltpu.SemaphoreType.DMA((2,)),
            pltpu.SemaphoreType.DMA((2,)),
        ],
    )
    def k(tab_hbm, idx_hbm, out_hbm, idx_s, buf_v, isem, gsem, wsem):
        wid = lax.axis_index("s") * _NC + lax.axis_index("c")
        base = wid * _ROWS_PER_W
        pltpu.sync_copy(idx_hbm.at[pl.ds(base, _ROWS_PER_W)], idx_s)

        def fire(bi, s):
            vec = idx_s[pl.ds(bi * _BATCH, _BATCH)]
            return [
                pltpu.async_copy(
                    tab_hbm.at[pl.ds(pl.multiple_of(vec[j], 8), 8)],
                    buf_v.at[s, j], gsem.at[s])
                for j in range(_BATCH)
            ]

        def writeback(bi, s):
            dst = pl.ds(base + bi * _BATCH, _BATCH)
            return pltpu.async_copy(buf_v.at[s], out_hbm.at[dst], wsem.at[s])

        g = {0: fire(0, 0)}
        wb = {}
        for bi in range(nbatch):
            s = bi % 2
            if bi + 1 < nbatch:
                if bi - 1 >= 0:
                    wb[bi - 1].wait()
                g[bi + 1] = fire(bi + 1, 1 - s)
            for c in g[bi]:
                c.wait()
            wb[bi] = writeback(bi, s)
        wb[nbatch - 2].wait()
        wb[nbatch - 1].wait()

    return k(table, idx8)


def _pick(rows8, sub):
    def sel(lo, hi, a, b):
        return jnp.where(sub < ((lo + hi + 1) // 2), a, b)

    q01 = sel(0, 1, rows8[:, 0, :], rows8[:, 1, :])
    q23 = sel(2, 3, rows8[:, 2, :], rows8[:, 3, :])
    q45 = sel(4, 5, rows8[:, 4, :], rows8[:, 5, :])
    q67 = sel(6, 7, rows8[:, 6, :], rows8[:, 7, :])
    h0 = jnp.where(sub < 2, q01, q23)
    h1 = jnp.where(sub < 6, q45, q67)
    return jnp.where(sub < 4, h0, h1).astype(jnp.float32)


def _mlp_body(u8_ref, i8_ref, us_ref, is_ref, w1u_ref, w1i_ref, b1_ref,
              w2_ref, b2_ref, wf_ref, bf_ref, o_ref):
    u = _pick(u8_ref[...], us_ref[...])
    it = _pick(i8_ref[...], is_ref[...])
    h1 = jnp.dot(u, w1u_ref[...], preferred_element_type=jnp.float32)
    h1 = h1 + jnp.dot(it, w1i_ref[...], preferred_element_type=jnp.float32)
    h1 = jnp.maximum(h1 + b1_ref[...], 0.0)
    h2 = jnp.dot(h1, w2_ref[...], preferred_element_type=jnp.float32)
    h2 = jnp.maximum(h2 + b2_ref[...], 0.0)
    z = jnp.dot(h2, wf_ref[...], preferred_element_type=jnp.float32)
    o_ref[...] = jax.nn.sigmoid(z + bf_ref[...])


def _mlp(u8, i8, usub, isub, W1, b1, W2, b2, Wf, bf, blk=512):
    n_blocks = _B // blk
    return pl.pallas_call(
        _mlp_body,
        grid=(n_blocks,),
        in_specs=[
            pl.BlockSpec((blk, 8, _D), lambda i: (i, 0, 0)),
            pl.BlockSpec((blk, 8, _D), lambda i: (i, 0, 0)),
            pl.BlockSpec((blk, 1), lambda i: (i, 0)),
            pl.BlockSpec((blk, 1), lambda i: (i, 0)),
            pl.BlockSpec((_D, _H1), lambda i: (0, 0)),
            pl.BlockSpec((_D, _H1), lambda i: (0, 0)),
            pl.BlockSpec((1, _H1), lambda i: (0, 0)),
            pl.BlockSpec((_H1, _H2), lambda i: (0, 0)),
            pl.BlockSpec((1, _H2), lambda i: (0, 0)),
            pl.BlockSpec((_H2, 1), lambda i: (0, 0)),
            pl.BlockSpec((1, 1), lambda i: (0, 0)),
        ],
        out_specs=pl.BlockSpec((blk, 1), lambda i: (i, 0)),
        out_shape=jax.ShapeDtypeStruct((_B, 1), jnp.float32),
    )(u8, i8, usub, isub, W1[:_D], W1[_D:], b1.reshape(1, _H1), W2,
      b2.reshape(1, _H2), Wf, bf.reshape(1, 1))


def kernel(users, items, user_table, item_table, W1, b1, W2, b2, Wf, bf):
    users = users.astype(jnp.int32)
    items = items.astype(jnp.int32)
    uidx8 = (users // 8) * 8
    iidx8 = (items // 8) * 8
    usub = (users % 8).reshape(_B, 1)
    isub = (items % 8).reshape(_B, 1)
    utb = user_table.astype(jnp.bfloat16)
    gu = _sc_gather(uidx8, utb)
    itb = item_table.astype(jnp.bfloat16)
    gi = _sc_gather(iidx8, itb)
    return _mlp(gu, gi, usub, isub, W1, b1, W2, b2, Wf, bf)


# speedup vs baseline: 3.8621x; 1.0080x over previous
import functools

import jax
import jax.numpy as jnp
from jax import lax
from jax.experimental import pallas as pl
from jax.experimental.pallas import tpu as pltpu
from jax.experimental.pallas import tpu_sc as plsc

_B = 16384
_D = 64
_H1 = 256
_H2 = 128
_V = 1000000

_NC = 2
_NS = 16
_NW = _NC * _NS
_ROWS_PER_W = _B // _NW
_BATCH = 64


def _sc_gather(idx8, table):
    mesh = plsc.VectorSubcoreMesh(core_axis_name="c", subcore_axis_name="s")
    nbatch = _ROWS_PER_W // _BATCH

    @functools.partial(
        pl.kernel,
        out_type=jax.ShapeDtypeStruct((_B, 8, _D), jnp.bfloat16),
        mesh=mesh,
        scratch_types=[
            pltpu.VMEM((_ROWS_PER_W,), jnp.int32),
            pltpu.VMEM((2, _BATCH, 8, _D), jnp.bfloat16),
            pltpu.SemaphoreType.DMA,
            pltpu.SemaphoreType.DMA((2,)),
            pltpu.SemaphoreType.DMA((2,)),
        ],
    )
    def k(tab_hbm, idx_hbm, out_hbm, idx_s, buf_v, isem, gsem, wsem):
        wid = lax.axis_index("s") * _NC + lax.axis_index("c")
        base = wid * _ROWS_PER_W
        pltpu.sync_copy(idx_hbm.at[pl.ds(base, _ROWS_PER_W)], idx_s)

        def fire(bi, s):
            copies = []
            for v in range(_BATCH // 16):
                vec = idx_s[pl.ds(bi * _BATCH + v * 16, 16)]
                for j in range(16):
                    copies.append(pltpu.async_copy(
                        tab_hbm.at[pl.ds(pl.multiple_of(vec[j], 8), 8)],
                        buf_v.at[s, v * 16 + j], gsem.at[s]))
            return copies

        def writeback(bi, s):
            dst = pl.ds(base + bi * _BATCH, _BATCH)
            return pltpu.async_copy(buf_v.at[s], out_hbm.at[dst], wsem.at[s])

        g = {0: fire(0, 0)}
        wb = {}
        for bi in range(nbatch):
            s = bi % 2
            if bi + 1 < nbatch:
                if bi - 1 >= 0:
                    wb[bi - 1].wait()
                g[bi + 1] = fire(bi + 1, 1 - s)
            for c in g[bi]:
                c.wait()
            wb[bi] = writeback(bi, s)
        wb[nbatch - 2].wait()
        wb[nbatch - 1].wait()

    return k(table, idx8)


def _pick(rows8, sub):
    def sel(lo, hi, a, b):
        return jnp.where(sub < ((lo + hi + 1) // 2), a, b)

    q01 = sel(0, 1, rows8[:, 0, :], rows8[:, 1, :])
    q23 = sel(2, 3, rows8[:, 2, :], rows8[:, 3, :])
    q45 = sel(4, 5, rows8[:, 4, :], rows8[:, 5, :])
    q67 = sel(6, 7, rows8[:, 6, :], rows8[:, 7, :])
    h0 = jnp.where(sub < 2, q01, q23)
    h1 = jnp.where(sub < 6, q45, q67)
    return jnp.where(sub < 4, h0, h1).astype(jnp.float32)


def _mlp_body(u8_ref, i8_ref, us_ref, is_ref, w1u_ref, w1i_ref, b1_ref,
              w2_ref, b2_ref, wf_ref, bf_ref, o_ref):
    u = _pick(u8_ref[...], us_ref[...])
    it = _pick(i8_ref[...], is_ref[...])
    h1 = jnp.dot(u, w1u_ref[...], preferred_element_type=jnp.float32)
    h1 = h1 + jnp.dot(it, w1i_ref[...], preferred_element_type=jnp.float32)
    h1 = jnp.maximum(h1 + b1_ref[...], 0.0)
    h2 = jnp.dot(h1, w2_ref[...], preferred_element_type=jnp.float32)
    h2 = jnp.maximum(h2 + b2_ref[...], 0.0)
    z = jnp.dot(h2, wf_ref[...], preferred_element_type=jnp.float32)
    o_ref[...] = jax.nn.sigmoid(z + bf_ref[...])


def _mlp(u8, i8, usub, isub, W1, b1, W2, b2, Wf, bf, blk=1024):
    n_blocks = _B // blk
    return pl.pallas_call(
        _mlp_body,
        grid=(n_blocks,),
        in_specs=[
            pl.BlockSpec((blk, 8, _D), lambda i: (i, 0, 0)),
            pl.BlockSpec((blk, 8, _D), lambda i: (i, 0, 0)),
            pl.BlockSpec((blk, 1), lambda i: (i, 0)),
            pl.BlockSpec((blk, 1), lambda i: (i, 0)),
            pl.BlockSpec((_D, _H1), lambda i: (0, 0)),
            pl.BlockSpec((_D, _H1), lambda i: (0, 0)),
            pl.BlockSpec((1, _H1), lambda i: (0, 0)),
            pl.BlockSpec((_H1, _H2), lambda i: (0, 0)),
            pl.BlockSpec((1, _H2), lambda i: (0, 0)),
            pl.BlockSpec((_H2, 1), lambda i: (0, 0)),
            pl.BlockSpec((1, 1), lambda i: (0, 0)),
        ],
        out_specs=pl.BlockSpec((blk, 1), lambda i: (i, 0)),
        out_shape=jax.ShapeDtypeStruct((_B, 1), jnp.float32),
    )(u8, i8, usub, isub, W1[:_D], W1[_D:], b1.reshape(1, _H1), W2,
      b2.reshape(1, _H2), Wf, bf.reshape(1, 1))


def kernel(users, items, user_table, item_table, W1, b1, W2, b2, Wf, bf):
    users = users.astype(jnp.int32)
    items = items.astype(jnp.int32)
    uidx8 = (users // 8) * 8
    iidx8 = (items // 8) * 8
    usub = (users % 8).reshape(_B, 1)
    isub = (items % 8).reshape(_B, 1)
    utb = user_table.astype(jnp.bfloat16)
    gu = _sc_gather(uidx8, utb)
    itb = item_table.astype(jnp.bfloat16)
    gi = _sc_gather(iidx8, itb)
    return _mlp(gu, gi, usub, isub, W1, b1, W2, b2, Wf, bf)
